# baseline (speedup 1.0000x reference)
import functools

import jax
import jax.numpy as jnp
from jax import lax
from jax.experimental import pallas as pl
from jax.experimental.pallas import tpu as pltpu
from jax.experimental.pallas import tpu_sc as plsc

_NC = 2
_NS = 16
_NW = _NC * _NS
_L = 16

_G = 128
_C = 16
_B = 320
_NGROUPS = _B // _L
_NROWS = 4 * _B
_NSTREAMS = _NROWS // 128


def _interp_body(table_hbm, xs_hbm, ys_hbm, zs_hbm, out_hbm,
                 vbuf, idx_v, wt_v, g_v, o_v,
                 gsem0, gsem1, osem0, osem1, csem):
    wid = lax.axis_index("s") * _NC + lax.axis_index("c")
    n_chunks = xs_hbm.shape[0] // _B
    my_count = (n_chunks - wid + _NW - 1) // _NW
    gsems = (gsem0, gsem1)
    osems = (osem0, osem1)

    def chunk_base(j):
        return (wid + _NW * j) * _B

    def stage_a(j, b):
        base = chunk_base(j)
        cx = pltpu.async_copy(xs_hbm.at[pl.ds(base, _B)], vbuf.at[0], csem)
        cy = pltpu.async_copy(ys_hbm.at[pl.ds(base, _B)], vbuf.at[1], csem)
        cz = pltpu.async_copy(zs_hbm.at[pl.ds(base, _B)], vbuf.at[2], csem)
        cx.wait()
        cy.wait()
        cz.wait()

        def group_idx_body(i, _):
            off = i * _L
            x = vbuf[0, pl.ds(off, _L)]
            y = vbuf[1, pl.ds(off, _L)]
            z = vbuf[2, pl.ds(off, _L)]
            half = jnp.float32(0.5 * (_G - 1))
            px = jnp.clip((x + 1.0) * half, 0.0, jnp.float32(_G - 1))
            py = jnp.clip((y + 1.0) * half, 0.0, jnp.float32(_G - 1))
            pz = jnp.clip((z + 1.0) * half, 0.0, jnp.float32(_G - 1))
            ix0 = jnp.minimum(px.astype(jnp.int32), _G - 2)
            iy0 = jnp.minimum(py.astype(jnp.int32), _G - 2)
            iz0 = jnp.minimum(pz.astype(jnp.int32), _G - 2)
            wx = px - ix0.astype(jnp.float32)
            wy = py - iy0.astype(jnp.float32)
            wz = pz - iz0.astype(jnp.float32)
            wx0 = 1.0 - wx
            wy0 = 1.0 - wy
            wz0 = 1.0 - wz

            zy00 = iz0 * (_G * _G) + iy0 * _G + ix0
            zy01 = zy00 + _G
            zy10 = zy00 + (_G * _G)
            zy11 = zy10 + _G
            idx4 = (zy00, zy01, zy10, zy11)

            t00 = wz0 * wy0
            t01 = wz0 * wy
            t10 = wz * wy0
            t11 = wz * wy
            wt8 = (t00 * wx0, t00 * wx, t01 * wx0, t01 * wx,
                   t10 * wx0, t10 * wx, t11 * wx0, t11 * wx)

            for p4 in range(4):
                idx_v[b, pl.ds(p4 * _B + off, _L)] = idx4[p4]
            for c in range(8):
                wt_v[b, c, pl.ds(off, _L)] = wt8[c]
            return 0

        lax.fori_loop(0, _NGROUPS, group_idx_body, 0)

        for s in range(_NSTREAMS):
            pltpu.make_async_copy(
                table_hbm.at[idx_v.at[b, pl.ds(s * 128, 128)]],
                g_v.at[b, pl.ds(s * 128, 128)], gsems[b]).start()

    def stage_c(j, b):
        @pl.when(j >= 2)
        def _():
            pltpu.make_async_copy(
                o_v.at[b], out_hbm.at[pl.ds(chunk_base(j - 2), _B), :],
                osems[b]).wait()

        pltpu.make_async_copy(
            table_hbm.at[idx_v.at[b]], g_v.at[b], gsems[b]).wait()

        topmask = jnp.int32(-65536)

        def group_sum_body(i, _):
            off = i * _L
            wv = [wt_v[b, c, pl.ds(off, _L)] for c in range(8)]
            for q in range(_L):
                p = off + q
                acc = None
                for p4 in range(4):
                    lanes = plsc.bitcast(g_v[b, p4 * _B + p, :], jnp.int32)
                    lo = plsc.bitcast(lanes << 16, jnp.float32)
                    hi = plsc.bitcast(lanes & topmask, jnp.float32)
                    term = lo * wv[2 * p4][q] + hi * wv[2 * p4 + 1][q]
                    acc = term if acc is None else acc + term
                o_v[b, p, :] = acc
            return 0

        lax.fori_loop(0, _NGROUPS, group_sum_body, 0)

        pltpu.make_async_copy(
            o_v.at[b], out_hbm.at[pl.ds(chunk_base(j), _B), :],
            osems[b]).start()

    stage_a(0, 0)

    def pair_body(j0, _):
        for b in range(2):
            j = j0 * 2 + b

            @pl.when(j + 1 < my_count)
            def _():
                stage_a(j + 1, 1 - b)

            @pl.when(j < my_count)
            def _():
                stage_c(j, b)
        return 0

    lax.fori_loop(0, (my_count + 1) // 2, pair_body, 0)

    m1 = my_count - 1
    for b in range(2):
        jlast = m1 - ((m1 - b) % 2)

        @pl.when(jlast >= 0)
        def _():
            pltpu.make_async_copy(
                o_v.at[b], out_hbm.at[pl.ds(chunk_base(jlast), _B), :],
                osems[b]).wait()


def _make_sc_interp(n_points):
    mesh = plsc.VectorSubcoreMesh(core_axis_name="c", subcore_axis_name="s")
    return functools.partial(
        pl.kernel,
        mesh=mesh,
        out_type=jax.ShapeDtypeStruct((n_points, _C), jnp.float32),
        scratch_types=[
            pltpu.VMEM((3, _B), jnp.float32),
            pltpu.VMEM((2, _NROWS), jnp.int32),
            pltpu.VMEM((2, 8, _B), jnp.float32),
            pltpu.VMEM((2, _NROWS, 2 * _C), jnp.bfloat16),
            pltpu.VMEM((2, _B, _C), jnp.float32),
            pltpu.SemaphoreType.DMA,
            pltpu.SemaphoreType.DMA,
            pltpu.SemaphoreType.DMA,
            pltpu.SemaphoreType.DMA,
            pltpu.SemaphoreType.DMA,
        ],
        compiler_params=pltpu.CompilerParams(
            use_tc_tiling_on_sc=False, needs_layout_passes=False),
    )(_interp_body)


def kernel(grid_features, vertices, point_features):
    n = vertices.shape[0]
    t = jnp.transpose(grid_features[0], (1, 2, 3, 0))
    tn = jnp.concatenate([t[:, :, 1:, :], t[:, :, -1:, :]], axis=2)
    table = jnp.stack([t, tn], axis=-1).astype(jnp.bfloat16).reshape(
        _G * _G * _G, 2 * _C)
    vt = vertices.T
    sampled = _make_sc_interp(n)(table, vt[0], vt[1], vt[2])
    return jnp.concatenate([point_features, sampled], axis=-1)


# speedup vs baseline: 4.0653x; 4.0653x over previous
import functools

import jax
import jax.numpy as jnp
from jax import lax
from jax.experimental import pallas as pl
from jax.experimental.pallas import tpu as pltpu
from jax.experimental.pallas import tpu_sc as plsc

_NC = 2
_NS = 16
_NW = _NC * _NS
_L = 16

_G = 128
_C = 16
_B = 320
_NGROUPS = _B // _L
_NROWS = 4 * _B
_NSTREAMS = _NROWS // 128


def _interp_body(table_hbm, xs_hbm, ys_hbm, zs_hbm, out_hbm,
                 vbuf, idx_v, wt_v, g_v, o_v,
                 gsem0, gsem1, osem0, osem1, csem):
    wid = lax.axis_index("s") * _NC + lax.axis_index("c")
    n_chunks = xs_hbm.shape[0] // _B
    my_count = (n_chunks - wid + _NW - 1) // _NW
    gsems = (gsem0, gsem1)
    osems = (osem0, osem1)

    def chunk_base(j):
        return (wid + _NW * j) * _B

    def stage_a(j, b):
        base = chunk_base(j)
        cx = pltpu.async_copy(xs_hbm.at[pl.ds(base, _B)], vbuf.at[0], csem)
        cy = pltpu.async_copy(ys_hbm.at[pl.ds(base, _B)], vbuf.at[1], csem)
        cz = pltpu.async_copy(zs_hbm.at[pl.ds(base, _B)], vbuf.at[2], csem)
        cx.wait()
        cy.wait()
        cz.wait()

        def group_idx_body(i, _):
            off = i * _L
            x = vbuf[0, pl.ds(off, _L)]
            y = vbuf[1, pl.ds(off, _L)]
            z = vbuf[2, pl.ds(off, _L)]
            half = jnp.float32(0.5 * (_G - 1))
            px = jnp.clip((x + 1.0) * half, 0.0, jnp.float32(_G - 1))
            py = jnp.clip((y + 1.0) * half, 0.0, jnp.float32(_G - 1))
            pz = jnp.clip((z + 1.0) * half, 0.0, jnp.float32(_G - 1))
            ix0 = jnp.minimum(px.astype(jnp.int32), _G - 2)
            iy0 = jnp.minimum(py.astype(jnp.int32), _G - 2)
            iz0 = jnp.minimum(pz.astype(jnp.int32), _G - 2)
            wx = px - ix0.astype(jnp.float32)
            wy = py - iy0.astype(jnp.float32)
            wz = pz - iz0.astype(jnp.float32)
            wx0 = 1.0 - wx
            wy0 = 1.0 - wy
            wz0 = 1.0 - wz

            zy00 = iz0 * (_G * _G) + iy0 * _G + ix0
            zy01 = zy00 + _G
            zy10 = zy00 + (_G * _G)
            zy11 = zy10 + _G
            idx4 = (zy00, zy01, zy10, zy11)

            t00 = wz0 * wy0
            t01 = wz0 * wy
            t10 = wz * wy0
            t11 = wz * wy
            wt8 = (t00 * wx0, t00 * wx, t01 * wx0, t01 * wx,
                   t10 * wx0, t10 * wx, t11 * wx0, t11 * wx)

            for p4 in range(4):
                idx_v[b, pl.ds(p4 * _B + off, _L)] = idx4[p4]
            for c in range(8):
                wt_v[b, c, pl.ds(off, _L)] = wt8[c]
            return 0

        lax.fori_loop(0, _NGROUPS, group_idx_body, 0)

        for s in range(_NSTREAMS):
            pltpu.make_async_copy(
                table_hbm.at[idx_v.at[b, pl.ds(s * 128, 128)]],
                g_v.at[b, pl.ds(s * 128, 128)], gsems[b]).start()

    def stage_c(j, b):
        @pl.when(j >= 2)
        def _():
            pltpu.make_async_copy(
                o_v.at[b], out_hbm.at[pl.ds(chunk_base(j - 2), _B), :],
                osems[b]).wait()

        pltpu.make_async_copy(
            table_hbm.at[idx_v.at[b]], g_v.at[b], gsems[b]).wait()

        topmask = jnp.int32(-65536)

        def group_sum_body(i, _):
            off = i * _L
            wv = [wt_v[b, c, pl.ds(off, _L)] for c in range(8)]
            for q in range(_L):
                p = off + q
                acc = None
                for p4 in range(4):
                    lanes = plsc.bitcast(g_v[b, p4 * _B + p, :], jnp.int32)
                    lo = plsc.bitcast(lanes << 16, jnp.float32)
                    hi = plsc.bitcast(lanes & topmask, jnp.float32)
                    term = lo * wv[2 * p4][q] + hi * wv[2 * p4 + 1][q]
                    acc = term if acc is None else acc + term
                o_v[b, p, :] = acc
            return 0

        lax.fori_loop(0, _NGROUPS, group_sum_body, 0)

        pltpu.make_async_copy(
            o_v.at[b], out_hbm.at[pl.ds(chunk_base(j), _B), :],
            osems[b]).start()

    stage_a(0, 0)

    def pair_body(j0, _):
        for b in range(2):
            j = j0 * 2 + b

            @pl.when(j + 1 < my_count)
            def _():
                stage_a(j + 1, 1 - b)

            @pl.when(j < my_count)
            def _():
                stage_c(j, b)
        return 0

    lax.fori_loop(0, (my_count + 1) // 2, pair_body, 0)

    m1 = my_count - 1
    for b in range(2):
        jlast = m1 - ((m1 - b) % 2)

        @pl.when(jlast >= 0)
        def _():
            pltpu.make_async_copy(
                o_v.at[b], out_hbm.at[pl.ds(chunk_base(jlast), _B), :],
                osems[b]).wait()


def _make_sc_interp(n_points):
    mesh = plsc.VectorSubcoreMesh(core_axis_name="c", subcore_axis_name="s")
    return functools.partial(
        pl.kernel,
        mesh=mesh,
        out_type=jax.ShapeDtypeStruct((n_points, _C), jnp.float32),
        scratch_types=[
            pltpu.VMEM((3, _B), jnp.float32),
            pltpu.VMEM((2, _NROWS), jnp.int32),
            pltpu.VMEM((2, 8, _B), jnp.float32),
            pltpu.VMEM((2, _NROWS, _C), jnp.float32),
            pltpu.VMEM((2, _B, _C), jnp.float32),
            pltpu.SemaphoreType.DMA,
            pltpu.SemaphoreType.DMA,
            pltpu.SemaphoreType.DMA,
            pltpu.SemaphoreType.DMA,
            pltpu.SemaphoreType.DMA,
        ],
        compiler_params=pltpu.CompilerParams(
            use_tc_tiling_on_sc=False, needs_layout_passes=False),
    )(_interp_body)


def kernel(grid_features, vertices, point_features):
    n = vertices.shape[0]
    gi = jax.lax.bitcast_convert_type(grid_features[0], jnp.int32)
    gr = gi + jnp.int32(0x8000)
    gn = jnp.concatenate([gr[:, :, :, 1:], gr[:, :, :, -1:]], axis=3)
    packed = jnp.bitwise_or(
        jnp.bitwise_and(jnp.right_shift(gr, 16), jnp.int32(0xFFFF)),
        jnp.bitwise_and(gn, jnp.int32(-65536)))
    packedf = jax.lax.bitcast_convert_type(packed, jnp.float32)
    table = jnp.transpose(packedf, (1, 2, 3, 0)).reshape(_G * _G * _G, _C)
    vt = vertices.T
    sampled = _make_sc_interp(n)(table, vt[0], vt[1], vt[2])
    return jnp.concatenate([point_features, sampled], axis=-1)


# speedup vs baseline: 4.9961x; 1.2290x over previous
import functools

import jax
import jax.numpy as jnp
from jax import lax
from jax.experimental import pallas as pl
from jax.experimental.pallas import tpu as pltpu
from jax.experimental.pallas import tpu_sc as plsc

_NC = 2
_NS = 16
_NW = _NC * _NS
_L = 16

_G = 128
_C = 16
_B = 320
_NGROUPS = _B // _L
_NROWS = 8 * _B
_NSTREAMS = _NROWS // 128


def _interp_body(table_hbm, xs_hbm, ys_hbm, zs_hbm, out_hbm,
                 vbuf, idx_v, wt_v, g_v, o_v,
                 gsem0, gsem1, osem0, osem1, csem):
    wid = lax.axis_index("s") * _NC + lax.axis_index("c")
    n_chunks = xs_hbm.shape[0] // _B
    my_count = (n_chunks - wid + _NW - 1) // _NW
    gsems = (gsem0, gsem1)
    osems = (osem0, osem1)

    def chunk_base(j):
        return (wid + _NW * j) * _B

    def stage_a(j, b):
        base = chunk_base(j)
        cx = pltpu.async_copy(xs_hbm.at[pl.ds(base, _B)], vbuf.at[0], csem)
        cy = pltpu.async_copy(ys_hbm.at[pl.ds(base, _B)], vbuf.at[1], csem)
        cz = pltpu.async_copy(zs_hbm.at[pl.ds(base, _B)], vbuf.at[2], csem)
        cx.wait()
        cy.wait()
        cz.wait()

        def group_idx_body(i, _):
            off = i * _L
            x = vbuf[0, pl.ds(off, _L)]
            y = vbuf[1, pl.ds(off, _L)]
            z = vbuf[2, pl.ds(off, _L)]
            half = jnp.float32(0.5 * (_G - 1))
            px = jnp.clip((x + 1.0) * half, 0.0, jnp.float32(_G - 1))
            py = jnp.clip((y + 1.0) * half, 0.0, jnp.float32(_G - 1))
            pz = jnp.clip((z + 1.0) * half, 0.0, jnp.float32(_G - 1))
            ix0 = jnp.minimum(px.astype(jnp.int32), _G - 2)
            iy0 = jnp.minimum(py.astype(jnp.int32), _G - 2)
            iz0 = jnp.minimum(pz.astype(jnp.int32), _G - 2)
            wx = px - ix0.astype(jnp.float32)
            wy = py - iy0.astype(jnp.float32)
            wz = pz - iz0.astype(jnp.float32)
            wx0 = 1.0 - wx
            wy0 = 1.0 - wy
            wz0 = 1.0 - wz

            zy00 = iz0 * (_G * _G) + iy0 * _G
            zy01 = zy00 + _G
            zy10 = zy00 + (_G * _G)
            zy11 = zy10 + _G
            ix1 = ix0 + 1
            idx8 = (zy00 + ix0, zy00 + ix1, zy01 + ix0, zy01 + ix1,
                    zy10 + ix0, zy10 + ix1, zy11 + ix0, zy11 + ix1)

            t00 = wz0 * wy0
            t01 = wz0 * wy
            t10 = wz * wy0
            t11 = wz * wy
            wt8 = (t00 * wx0, t00 * wx, t01 * wx0, t01 * wx,
                   t10 * wx0, t10 * wx, t11 * wx0, t11 * wx)

            for c in range(8):
                idx_v[b, pl.ds(c * _B + off, _L)] = idx8[c]
                wt_v[b, c, pl.ds(off, _L)] = wt8[c]
            return 0

        lax.fori_loop(0, _NGROUPS, group_idx_body, 0)

        for s in range(_NSTREAMS):
            pltpu.make_async_copy(
                table_hbm.at[idx_v.at[b, pl.ds(s * 128, 128)]],
                g_v.at[b, pl.ds(s * 128, 128)], gsems[b]).start()

    def stage_c(j, b):
        @pl.when(j >= 2)
        def _():
            pltpu.make_async_copy(
                o_v.at[b], out_hbm.at[pl.ds(chunk_base(j - 2), _B), :],
                osems[b]).wait()

        pltpu.make_async_copy(
            table_hbm.at[idx_v.at[b]], g_v.at[b], gsems[b]).wait()

        def group_sum_body(i, _):
            off = i * _L
            wv = [wt_v[b, c, pl.ds(off, _L)] for c in range(8)]
            for q in range(_L):
                p = off + q
                acc = g_v[b, 0 * _B + p, :] * wv[0][q]
                for c in range(1, 8):
                    acc = acc + g_v[b, c * _B + p, :] * wv[c][q]
                o_v[b, p, :] = acc
            return 0

        lax.fori_loop(0, _NGROUPS, group_sum_body, 0)

        pltpu.make_async_copy(
            o_v.at[b], out_hbm.at[pl.ds(chunk_base(j), _B), :],
            osems[b]).start()

    stage_a(0, 0)

    def pair_body(j0, _):
        for b in range(2):
            j = j0 * 2 + b

            @pl.when(j + 1 < my_count)
            def _():
                stage_a(j + 1, 1 - b)

            @pl.when(j < my_count)
            def _():
                stage_c(j, b)
        return 0

    lax.fori_loop(0, (my_count + 1) // 2, pair_body, 0)

    m1 = my_count - 1
    for b in range(2):
        jlast = m1 - ((m1 - b) % 2)

        @pl.when(jlast >= 0)
        def _():
            pltpu.make_async_copy(
                o_v.at[b], out_hbm.at[pl.ds(chunk_base(jlast), _B), :],
                osems[b]).wait()


def _make_sc_interp(n_points):
    mesh = plsc.VectorSubcoreMesh(core_axis_name="c", subcore_axis_name="s")
    return functools.partial(
        pl.kernel,
        mesh=mesh,
        out_type=jax.ShapeDtypeStruct((n_points, _C), jnp.float32),
        scratch_types=[
            pltpu.VMEM((3, _B), jnp.float32),
            pltpu.VMEM((2, _NROWS), jnp.int32),
            pltpu.VMEM((2, 8, _B), jnp.float32),
            pltpu.VMEM((2, _NROWS, _C), jnp.float32),
            pltpu.VMEM((2, _B, _C), jnp.float32),
            pltpu.SemaphoreType.DMA,
            pltpu.SemaphoreType.DMA,
            pltpu.SemaphoreType.DMA,
            pltpu.SemaphoreType.DMA,
            pltpu.SemaphoreType.DMA,
        ],
        compiler_params=pltpu.CompilerParams(
            use_tc_tiling_on_sc=False, needs_layout_passes=False),
    )(_interp_body)


def kernel(grid_features, vertices, point_features):
    n = vertices.shape[0]
    table = jnp.transpose(grid_features[0], (1, 2, 3, 0)).reshape(_G * _G * _G, _C)
    vt = vertices.T
    sampled = _make_sc_interp(n)(table, vt[0], vt[1], vt[2])
    return jnp.concatenate([point_features, sampled], axis=-1)
